# baseline (speedup 1.0000x reference)
import functools

import jax
import jax.numpy as jnp
from jax import lax
from jax.experimental import pallas as pl
from jax.experimental.pallas import tpu as pltpu
from jax.experimental.pallas import tpu_sc as plsc

N = 10000
E = 320000
C = 32
K = 10
ALPHA = 0.1

NP_ = 10240
HALF = NP_ // 2
CH = NP_ // 16
UPT = HALF // 16
NTILES = 32
EPT = E // NTILES
KB = 128
EBP = 80
CHUNK = EBP * KB
CAPB = 144
PAD_ROWS = NP_ - N

_MESH = plsc.VectorSubcoreMesh(core_axis_name="c", subcore_axis_name="s")
_SC_PARAMS = pltpu.CompilerParams(use_tc_tiling_on_sc=False,
                                  needs_layout_passes=False)


def _part_body(srcf_hbm, dstf_hbm, padpat_hbm, srcP_hbm, dstP_hbm, cnt_hbm,
               sbuf, dbuf, srcl, dstl, cntb):
    c = lax.axis_index("c")
    s = lax.axis_index("s")
    lo = jnp.full((16,), c * HALF, dtype=jnp.int32)

    pltpu.sync_copy(padpat_hbm, srcl)
    pltpu.sync_copy(padpat_hbm, dstl)

    off = jnp.zeros((16,), dtype=jnp.int32)
    for half_id in range(2):
        o = s + 16 * half_id
        pltpu.sync_copy(srcf_hbm.at[o], sbuf)
        pltpu.sync_copy(dstf_hbm.at[o], dbuf)

        def scan(i, off):
            sv = sbuf[pl.ds(i * 16, 16)]
            dv = dbuf[pl.ds(i * 16, 16)]
            dl = dv - lo
            m = (dl >= 0) & (dl < HALF)
            pos = off + plsc.cumsum(jnp.where(m, 1, 0).astype(jnp.int32)) - 1
            row = lax.shift_right_logical(pos, 7)
            col = lax.bitwise_and(pos, 127)
            plsc.store_scatter(srcl, [row, col], sv, mask=m)
            plsc.store_scatter(dstl, [row, col], dv, mask=m)
            return off + plsc.all_reduce_population_count(m)

        off = lax.fori_loop(0, CHUNK // 16, scan, off)

    cntb[...] = off
    pltpu.sync_copy(srcl, srcP_hbm.at[c, s])
    pltpu.sync_copy(dstl, dstP_hbm.at[c, s])
    pltpu.sync_copy(cntb, cnt_hbm.at[c, s])


_sc_part = functools.partial(
    pl.kernel,
    out_type=(
        jax.ShapeDtypeStruct((2, 16, CAPB, KB), jnp.int32),
        jax.ShapeDtypeStruct((2, 16, CAPB, KB), jnp.int32),
        jax.ShapeDtypeStruct((2, 16, 16), jnp.int32),
    ),
    mesh=_MESH,
    compiler_params=_SC_PARAMS,
    scratch_types=[
        pltpu.VMEM((CHUNK,), jnp.int32),
        pltpu.VMEM((CHUNK,), jnp.int32),
        pltpu.VMEM((CAPB, KB), jnp.int32),
        pltpu.VMEM((CAPB, KB), jnp.int32),
        pltpu.VMEM((16,), jnp.int32),
    ],
)(_part_body)


_MAGIC = 0x5CA77E00


def _rounds_body(nrounds, g0_hbm, d29_hbm, zz_hbm, srcP_hbm, dstP_hbm,
                 cnt_hbm, gx_hbm, fl_hbm, out_hbm, src_v, dst_v, rows0, rows1,
                 dbuf2, zbuf2, abuf, gbuf, cntb, flagb,
                 agg_sh, sem0, sem1):
    c = lax.axis_index("c")
    s = lax.axis_index("s")
    peer = 1 - c
    myrow = c * HALF + s * UPT
    peerrow = peer * HALF + s * UPT
    srow = s * CH

    pltpu.sync_copy(srcP_hbm.at[c, s], src_v)
    pltpu.sync_copy(dstP_hbm.at[c, s], dst_v)
    pltpu.sync_copy(cnt_hbm.at[c, s], cntb)
    pltpu.sync_copy(d29_hbm.at[pl.ds(myrow, UPT)], dbuf2)
    pltpu.sync_copy(zz_hbm.at[pl.ds(myrow, UPT)], zbuf2)
    pltpu.sync_copy(g0_hbm.at[pl.ds(srow, CH)], gx_hbm.at[0, pl.ds(srow, CH)])

    pltpu.sync_copy(g0_hbm.at[pl.ds(myrow, UPT)], agg_sh.at[pl.ds(myrow, UPT)])
    cnt = cntb[...][0]
    nsteps = lax.shift_right_logical(cnt + 2 * KB - 1, 8)

    def round_body(r, carry):
        @pl.when(r > 0)
        def _():
            def poll_cond(fv):
                return fv != _MAGIC + r

            def poll(fv):
                pltpu.sync_copy(fl_hbm.at[peer, s], flagb)
                return flagb[...][0]

            lax.while_loop(poll_cond, poll, jnp.int32(0))

        plsc.subcore_barrier()
        par = lax.bitwise_and(r, 1)
        gsrc = gx_hbm.at[par]

        @pl.when(nsteps > 0)
        def _():
            pltpu.async_copy(gsrc.at[src_v.at[0]], rows0, sem0)

        def step(jj, carry):
            j = 2 * jj
            pltpu.make_async_copy(gsrc.at[src_v.at[j]], rows0, sem0).wait()
            pltpu.async_copy(gsrc.at[src_v.at[j + 1]], rows1, sem1)
            pltpu.sync_copy(rows0, agg_sh.at[dst_v.at[j]], add=True)
            pltpu.make_async_copy(gsrc.at[src_v.at[j + 1]], rows1, sem1).wait()

            @pl.when(jj < nsteps - 1)
            def _():
                pltpu.async_copy(gsrc.at[src_v.at[j + 2]], rows0, sem0)

            pltpu.sync_copy(rows1, agg_sh.at[dst_v.at[j + 1]], add=True)
            return carry

        lax.fori_loop(0, nsteps, step, 0)
        plsc.subcore_barrier()

        pltpu.sync_copy(agg_sh.at[pl.ds(myrow, UPT)], abuf)

        def upd(i, carry):
            for h in (0, 16):
                gbuf[i, pl.ds(h, 16)] = (
                    dbuf2[i, pl.ds(h, 16)] * abuf[i, pl.ds(h, 16)]
                    + zbuf2[i, pl.ds(h, 16)])
            return carry

        lax.fori_loop(0, UPT, upd, 0)
        pltpu.sync_copy(gbuf, agg_sh.at[pl.ds(myrow, UPT)])

        @pl.when(r < nrounds - 1)
        def _():
            pltpu.sync_copy(gbuf,
                            gx_hbm.at[1 - par, pl.ds(myrow, UPT)])
            flagb[...] = jnp.full((16,), _MAGIC + 1 + r, dtype=jnp.int32)
            pltpu.sync_copy(flagb, fl_hbm.at[c, s])

        return carry

    lax.fori_loop(0, nrounds, round_body, 0)
    pltpu.sync_copy(gbuf, out_hbm.at[pl.ds(myrow, UPT)])


def _make_rounds(nrounds):
    return functools.partial(
        pl.kernel,
        out_type=jax.ShapeDtypeStruct((NP_, C), jnp.float32),
        mesh=_MESH,
        compiler_params=_SC_PARAMS,
        scratch_types=[
            pltpu.VMEM((CAPB, KB), jnp.int32),
            pltpu.VMEM((CAPB, KB), jnp.int32),
            pltpu.VMEM((KB, C), jnp.float32),
            pltpu.VMEM((KB, C), jnp.float32),
            pltpu.VMEM((UPT, C), jnp.float32),
            pltpu.VMEM((UPT, C), jnp.float32),
            pltpu.VMEM((UPT, C), jnp.float32),
            pltpu.VMEM((UPT, C), jnp.float32),
            pltpu.VMEM((16,), jnp.int32),
            pltpu.VMEM((16,), jnp.int32),
            pltpu.VMEM_SHARED((NP_, C), jnp.float32),
            pltpu.SemaphoreType.DMA,
            pltpu.SemaphoreType.DMA,
        ],
    )(functools.partial(_rounds_body, nrounds))


_sc_deg = _make_rounds(1)
_sc_rounds = _make_rounds(K)


def _mlp_body(x_ref, w1_ref, b1_ref, w2_ref, b2_ref, z_ref):
    h = jnp.maximum(
        jax.lax.dot_general(x_ref[...], w1_ref[...], (((1,), (0,)), ((), ())),
                            preferred_element_type=jnp.float32) + b1_ref[...],
        0.0)
    z_ref[...] = jax.lax.dot_general(h, w2_ref[...], (((1,), (0,)), ((), ())),
                                     preferred_element_type=jnp.float32) + b2_ref[...]


def _mlp(x, W1, b1, W2, b2):
    nblk = 10
    rows = N // nblk
    return pl.pallas_call(
        _mlp_body,
        grid=(nblk,),
        in_specs=[
            pl.BlockSpec((rows, 128), lambda i: (i, 0)),
            pl.BlockSpec((128, 256), lambda i: (0, 0)),
            pl.BlockSpec((1, 256), lambda i: (0, 0)),
            pl.BlockSpec((256, C), lambda i: (0, 0)),
            pl.BlockSpec((1, C), lambda i: (0, 0)),
        ],
        out_specs=pl.BlockSpec((rows, C), lambda i: (i, 0)),
        out_shape=jax.ShapeDtypeStruct((N, C), jnp.float32),
    )(x, W1, b1.reshape(1, 256), W2, b2.reshape(1, C))


def _prep_body(deg_ref, zp_ref, d29_ref, zz_ref, g0_ref, sq_ref):
    deg = deg_ref[...]
    dis = jax.lax.rsqrt(deg)
    zp = zp_ref[...]
    d29_ref[...] = (1.0 - ALPHA) / deg
    zz_ref[...] = ALPHA * dis * zp
    g0_ref[...] = dis * zp
    sq_ref[...] = jnp.sqrt(deg)


def _prep(deg, zp):
    shp = jax.ShapeDtypeStruct((NP_, C), jnp.float32)
    return pl.pallas_call(
        _prep_body,
        out_shape=(shp, shp, shp, shp),
    )(deg, zp)


def _final_body(g_ref, sq_ref, lp_ref, h_ref):
    h = g_ref[...] * sq_ref[...]
    m = jnp.max(h, axis=1, keepdims=True)
    e = jnp.exp(h - m)
    ssum = jnp.sum(e, axis=1, keepdims=True)
    lp_ref[...] = (h - m) - jnp.log(ssum)
    h_ref[...] = h


def _final(g10, sqf):
    shp = jax.ShapeDtypeStruct((N, C), jnp.float32)
    return pl.pallas_call(
        _final_body,
        out_shape=(shp, shp),
    )(g10[:N], sqf[:N])


def kernel(x, edge_index, W1, b1, W2, b2):
    src = edge_index[0].reshape(NTILES, EPT)
    dst = edge_index[1].reshape(NTILES, EPT)
    npad = CHUNK - EPT
    padidx = N + (jnp.arange(npad, dtype=jnp.int32) % PAD_ROWS)
    padblk = jnp.broadcast_to(padidx, (NTILES, npad))
    srcf = jnp.concatenate([src, padblk], axis=1)
    dstf = jnp.concatenate([dst, padblk], axis=1)
    padpat = (N + (jnp.arange(CAPB * KB, dtype=jnp.int32) % PAD_ROWS)
              ).reshape(CAPB, KB)

    zerosN = jnp.zeros((NP_, C), dtype=jnp.float32)
    onesN = jnp.ones((NP_, C), dtype=jnp.float32)

    srcP, dstP, cnts = _sc_part(srcf, dstf, padpat)
    z = _mlp(x, W1, b1, W2, b2)
    zp = jnp.pad(z, ((0, PAD_ROWS), (0, 0)))

    fl0 = jnp.zeros((2, 16, 16), jnp.int32) + (edge_index[0, 0] * 0)
    gx0 = jnp.zeros((2, NP_, C), jnp.float32)

    deg = _sc_deg(onesN, onesN, zerosN, srcP, dstP, cnts, gx0, fl0)
    d29f, zzf, g0, sqf = _prep(deg, zp)

    g10 = _sc_rounds(g0, d29f, zzf, srcP, dstP, cnts, gx0, fl0)
    return _final(g10, sqf)


# speedup vs baseline: 1.8540x; 1.8540x over previous
import functools

import jax
import jax.numpy as jnp
from jax import lax
from jax.experimental import pallas as pl
from jax.experimental.pallas import tpu as pltpu
from jax.experimental.pallas import tpu_sc as plsc

N = 10000
E = 320000
C = 32
K = 10
ALPHA = 0.1

NP_ = 10240
HALF = NP_ // 2
CH = NP_ // 16
UPT = HALF // 16
NTILES = 32
EPT = E // NTILES
KB = 128
EBP = 80
CHUNK = EBP * KB
CAPB = 144
PAD_ROWS = NP_ - N

_MESH = plsc.VectorSubcoreMesh(core_axis_name="c", subcore_axis_name="s")
_SC_PARAMS = pltpu.CompilerParams(use_tc_tiling_on_sc=False,
                                  needs_layout_passes=False)


def _part_body(srcf_hbm, dstf_hbm, padpat_hbm, srcP_hbm, dstP_hbm, cnt_hbm,
               sbuf, dbuf, srcl, dstl, cntb):
    c = lax.axis_index("c")
    s = lax.axis_index("s")
    lo = jnp.full((16,), c * HALF, dtype=jnp.int32)

    pltpu.sync_copy(padpat_hbm, srcl)
    pltpu.sync_copy(padpat_hbm, dstl)

    off = jnp.zeros((16,), dtype=jnp.int32)
    for half_id in range(2):
        o = s + 16 * half_id
        pltpu.sync_copy(srcf_hbm.at[o], sbuf)
        pltpu.sync_copy(dstf_hbm.at[o], dbuf)

        def scan(i, off):
            sv = sbuf[pl.ds(i * 16, 16)]
            dv = dbuf[pl.ds(i * 16, 16)]
            dl = dv - lo
            m = (dl >= 0) & (dl < HALF)
            pos = off + plsc.cumsum(jnp.where(m, 1, 0).astype(jnp.int32)) - 1
            row = lax.shift_right_logical(pos, 7)
            col = lax.bitwise_and(pos, 127)
            plsc.store_scatter(srcl, [row, col], sv, mask=m)
            plsc.store_scatter(dstl, [row, col], dv, mask=m)
            return off + plsc.all_reduce_population_count(m)

        off = lax.fori_loop(0, CHUNK // 16, scan, off)

    cntb[...] = off
    pltpu.sync_copy(srcl, srcP_hbm.at[c, s])
    pltpu.sync_copy(dstl, dstP_hbm.at[c, s])
    pltpu.sync_copy(cntb, cnt_hbm.at[c, s])


_sc_part = functools.partial(
    pl.kernel,
    out_type=(
        jax.ShapeDtypeStruct((2, 16, CAPB, KB), jnp.int32),
        jax.ShapeDtypeStruct((2, 16, CAPB, KB), jnp.int32),
        jax.ShapeDtypeStruct((2, 16, 16), jnp.int32),
    ),
    mesh=_MESH,
    compiler_params=_SC_PARAMS,
    scratch_types=[
        pltpu.VMEM((CHUNK,), jnp.int32),
        pltpu.VMEM((CHUNK,), jnp.int32),
        pltpu.VMEM((CAPB, KB), jnp.int32),
        pltpu.VMEM((CAPB, KB), jnp.int32),
        pltpu.VMEM((16,), jnp.int32),
    ],
)(_part_body)


_MAGIC = 0x5CA77E00


def _rounds_body(nrounds, g0_hbm, d29_hbm, zz_hbm, srcP_hbm, dstP_hbm,
                 cnt_hbm, gx_hbm, fl_hbm, out_hbm, src_v, dst_v, rows0, rows1,
                 dbuf2, zbuf2, abuf, gbuf, cntb, flagb,
                 g_sh, agg_sh, sem0, sem1):
    c = lax.axis_index("c")
    s = lax.axis_index("s")
    peer = 1 - c
    myrow = c * HALF + s * UPT
    peerrow = peer * HALF + s * UPT
    srow = s * CH

    pltpu.sync_copy(srcP_hbm.at[c, s], src_v)
    pltpu.sync_copy(dstP_hbm.at[c, s], dst_v)
    pltpu.sync_copy(cnt_hbm.at[c, s], cntb)
    pltpu.sync_copy(d29_hbm.at[pl.ds(myrow, UPT)], dbuf2)
    pltpu.sync_copy(zz_hbm.at[pl.ds(myrow, UPT)], zbuf2)
    pltpu.sync_copy(g0_hbm.at[pl.ds(srow, CH)], g_sh.at[pl.ds(srow, CH)])
    pltpu.sync_copy(g0_hbm.at[pl.ds(myrow, UPT)], agg_sh.at[pl.ds(myrow, UPT)])
    cnt = cntb[...][0]
    nsteps = lax.shift_right_logical(cnt + 2 * KB - 1, 8)
    plsc.subcore_barrier()

    def round_body(r, carry):
        @pl.when(nsteps > 0)
        def _():
            pltpu.async_copy(g_sh.at[src_v.at[0]], rows0, sem0)

        def step(jj, carry):
            j = 2 * jj
            pltpu.make_async_copy(g_sh.at[src_v.at[j]], rows0, sem0).wait()
            pltpu.async_copy(g_sh.at[src_v.at[j + 1]], rows1, sem1)
            pltpu.sync_copy(rows0, agg_sh.at[dst_v.at[j]], add=True)
            pltpu.make_async_copy(g_sh.at[src_v.at[j + 1]], rows1, sem1).wait()

            @pl.when(jj < nsteps - 1)
            def _():
                pltpu.async_copy(g_sh.at[src_v.at[j + 2]], rows0, sem0)

            pltpu.sync_copy(rows1, agg_sh.at[dst_v.at[j + 1]], add=True)
            return carry

        lax.fori_loop(0, nsteps, step, 0)
        plsc.subcore_barrier()

        pltpu.sync_copy(agg_sh.at[pl.ds(myrow, UPT)], abuf)

        def upd(i, carry):
            for h in (0, 16):
                gbuf[i, pl.ds(h, 16)] = (
                    dbuf2[i, pl.ds(h, 16)] * abuf[i, pl.ds(h, 16)]
                    + zbuf2[i, pl.ds(h, 16)])
            return carry

        lax.fori_loop(0, UPT, upd, 0)
        pltpu.sync_copy(gbuf, g_sh.at[pl.ds(myrow, UPT)])
        pltpu.sync_copy(gbuf, agg_sh.at[pl.ds(myrow, UPT)])

        @pl.when(r < nrounds - 1)
        def _():
            par = lax.bitwise_and(r + 1, 1)
            pltpu.sync_copy(gbuf, gx_hbm.at[par, c, pl.ds(s * UPT, UPT)])
            flagb[...] = jnp.full((16,), _MAGIC + 1 + r, dtype=jnp.int32)
            pltpu.sync_copy(flagb, fl_hbm.at[c, s])

            def poll_cond(fv):
                return fv != _MAGIC + 1 + r

            def poll(fv):
                pltpu.sync_copy(fl_hbm.at[peer, s], flagb)
                return flagb[...][0]

            lax.while_loop(poll_cond, poll, jnp.int32(0))
            pltpu.sync_copy(gx_hbm.at[par, peer, pl.ds(s * UPT, UPT)],
                            g_sh.at[pl.ds(peerrow, UPT)])

        plsc.subcore_barrier()
        return carry

    lax.fori_loop(0, nrounds, round_body, 0)
    pltpu.sync_copy(gbuf, out_hbm.at[pl.ds(myrow, UPT)])


def _make_rounds(nrounds):
    return functools.partial(
        pl.kernel,
        out_type=jax.ShapeDtypeStruct((NP_, C), jnp.float32),
        mesh=_MESH,
        compiler_params=_SC_PARAMS,
        scratch_types=[
            pltpu.VMEM((CAPB, KB), jnp.int32),
            pltpu.VMEM((CAPB, KB), jnp.int32),
            pltpu.VMEM((KB, C), jnp.float32),
            pltpu.VMEM((KB, C), jnp.float32),
            pltpu.VMEM((UPT, C), jnp.float32),
            pltpu.VMEM((UPT, C), jnp.float32),
            pltpu.VMEM((UPT, C), jnp.float32),
            pltpu.VMEM((UPT, C), jnp.float32),
            pltpu.VMEM((16,), jnp.int32),
            pltpu.VMEM((16,), jnp.int32),
            pltpu.VMEM_SHARED((NP_, C), jnp.float32),
            pltpu.VMEM_SHARED((NP_, C), jnp.float32),
            pltpu.SemaphoreType.DMA,
            pltpu.SemaphoreType.DMA,
        ],
    )(functools.partial(_rounds_body, nrounds))


_sc_deg = _make_rounds(1)
_sc_rounds = _make_rounds(K)


def _mlp_body(x_ref, w1_ref, b1_ref, w2_ref, b2_ref, z_ref):
    h = jnp.maximum(
        jax.lax.dot_general(x_ref[...], w1_ref[...], (((1,), (0,)), ((), ())),
                            preferred_element_type=jnp.float32) + b1_ref[...],
        0.0)
    z_ref[...] = jax.lax.dot_general(h, w2_ref[...], (((1,), (0,)), ((), ())),
                                     preferred_element_type=jnp.float32) + b2_ref[...]


def _mlp(x, W1, b1, W2, b2):
    nblk = 10
    rows = N // nblk
    return pl.pallas_call(
        _mlp_body,
        grid=(nblk,),
        in_specs=[
            pl.BlockSpec((rows, 128), lambda i: (i, 0)),
            pl.BlockSpec((128, 256), lambda i: (0, 0)),
            pl.BlockSpec((1, 256), lambda i: (0, 0)),
            pl.BlockSpec((256, C), lambda i: (0, 0)),
            pl.BlockSpec((1, C), lambda i: (0, 0)),
        ],
        out_specs=pl.BlockSpec((rows, C), lambda i: (i, 0)),
        out_shape=jax.ShapeDtypeStruct((N, C), jnp.float32),
    )(x, W1, b1.reshape(1, 256), W2, b2.reshape(1, C))


def _prep_body(deg_ref, zp_ref, d29_ref, zz_ref, g0_ref, sq_ref):
    deg = deg_ref[...]
    dis = jax.lax.rsqrt(deg)
    zp = zp_ref[...]
    d29_ref[...] = (1.0 - ALPHA) / deg
    zz_ref[...] = ALPHA * dis * zp
    g0_ref[...] = dis * zp
    sq_ref[...] = jnp.sqrt(deg)


def _prep(deg, zp):
    shp = jax.ShapeDtypeStruct((NP_, C), jnp.float32)
    return pl.pallas_call(
        _prep_body,
        out_shape=(shp, shp, shp, shp),
    )(deg, zp)


def _final_body(g_ref, sq_ref, lp_ref, h_ref):
    h = g_ref[...] * sq_ref[...]
    m = jnp.max(h, axis=1, keepdims=True)
    e = jnp.exp(h - m)
    ssum = jnp.sum(e, axis=1, keepdims=True)
    lp_ref[...] = (h - m) - jnp.log(ssum)
    h_ref[...] = h


def _final(g10, sqf):
    shp = jax.ShapeDtypeStruct((N, C), jnp.float32)
    return pl.pallas_call(
        _final_body,
        out_shape=(shp, shp),
    )(g10[:N], sqf[:N])


def kernel(x, edge_index, W1, b1, W2, b2):
    src = edge_index[0].reshape(NTILES, EPT)
    dst = edge_index[1].reshape(NTILES, EPT)
    npad = CHUNK - EPT
    padidx = N + (jnp.arange(npad, dtype=jnp.int32) % PAD_ROWS)
    padblk = jnp.broadcast_to(padidx, (NTILES, npad))
    srcf = jnp.concatenate([src, padblk], axis=1)
    dstf = jnp.concatenate([dst, padblk], axis=1)
    padpat = (N + (jnp.arange(CAPB * KB, dtype=jnp.int32) % PAD_ROWS)
              ).reshape(CAPB, KB)

    zerosN = jnp.zeros((NP_, C), dtype=jnp.float32)
    onesN = jnp.ones((NP_, C), dtype=jnp.float32)

    srcP, dstP, cnts = _sc_part(srcf, dstf, padpat)
    z = _mlp(x, W1, b1, W2, b2)
    zp = jnp.pad(z, ((0, PAD_ROWS), (0, 0)))

    fl0 = jnp.zeros((2, 16, 16), jnp.int32) + (edge_index[0, 0] * 0)
    gx0 = jnp.zeros((2, 2, HALF, C), jnp.float32)

    deg = _sc_deg(onesN, onesN, zerosN, srcP, dstP, cnts, gx0, fl0)
    d29f, zzf, g0, sqf = _prep(deg, zp)

    g10 = _sc_rounds(g0, d29f, zzf, srcP, dstP, cnts, gx0, fl0)
    return _final(g10, sqf)
